# baseline (speedup 1.0000x reference)
import functools

import jax
import jax.numpy as jnp
from jax import lax
from jax.experimental import pallas as pl
from jax.experimental.pallas import tpu as pltpu
from jax.experimental.pallas import tpu_sc as plsc

_VOCAB = 1001
_DIM = 64
_B = 16384
_L = 200
_VPAD = 1024

_NC, _NS, _LANES = 2, 16, 16
_NW = _NC * _NS
_GRP = _LANES
_NCHUNK = 4
_BC = _B // _NCHUNK
_NGROUPS = _BC // _GRP
_GPW = _NGROUPS // _NW


def _count_body(x_hbm, c_hbm, x_v, c_v, *, chunk):
    wid = lax.axis_index("s") * _NC + lax.axis_index("c")
    lane = lax.iota(jnp.int32, _LANES)
    ones = jnp.full((_LANES,), 1.0, jnp.float32)
    neg_ones = jnp.full((_LANES,), -1.0, jnp.float32)
    tiles_pw = _GPW * _GRP // 128

    for r in range(_GRP):
        for s in range(_VPAD // 128):
            @plsc.parallel_loop(0, 128, step=_LANES, unroll=8)
            def _zero(j, r=r, s=s):
                c_v[r, s, pl.ds(j, _LANES)] = jnp.zeros((_LANES,), jnp.float32)

    x_tile0 = chunk * (_BC // 128) + wid * tiles_pw
    pltpu.sync_copy(x_hbm.at[:, pl.ds(x_tile0, tiles_pw), :, :], x_v)

    def _scatter_pass(g_local, val):
        tile_l = lax.div(g_local, 8)
        c0 = lax.rem(g_local, 8) * _LANES

        @plsc.parallel_loop(0, _L, unroll=8)
        def _step(l):
            xv = x_v[jnp.right_shift(l, 3), tile_l,
                     jnp.bitwise_and(l, 7), pl.ds(c0, _LANES)]
            plsc.addupdate_scatter(
                c_v, [lane, jnp.right_shift(xv, 7), jnp.bitwise_and(xv, 127)],
                val)

    def _group(g_local, _):
        base = (wid * _GPW + g_local) * _GRP
        _scatter_pass(g_local, ones)
        pltpu.sync_copy(c_v, c_hbm.at[pl.ds(base, _GRP), :, :])
        _scatter_pass(g_local, neg_ones)
        return _
    lax.fori_loop(0, _GPW, _group, None)


@functools.cache
def _make_count(chunk):
    return pl.kernel(
        functools.partial(_count_body, chunk=chunk),
        out_type=jax.ShapeDtypeStruct((_BC, _VPAD // 128, 128), jnp.float32),
        mesh=plsc.VectorSubcoreMesh(core_axis_name="c", subcore_axis_name="s"),
        scratch_types=[
            pltpu.VMEM((_L // 8, _GPW * _GRP // 128, 8, 128), jnp.int32),
            pltpu.VMEM((_GRP, _VPAD // 128, 128), jnp.float32),
        ],
        compiler_params=pltpu.CompilerParams(
            needs_layout_passes=False, use_tc_tiling_on_sc=False),
    )


_BLK = 1024


def _proj_body(c_ref, t_ref, w_ref, b_ref, o_ref):
    pooled = jnp.zeros((_BLK, _DIM), jnp.float32)
    for s in range(_VPAD // 128):
        pooled += jnp.dot(c_ref[:, s, :], t_ref[pl.ds(s * 128, 128), :],
                          preferred_element_type=jnp.float32)
    pooled = pooled * (1.0 / _L)
    logits_t = lax.dot_general(
        w_ref[...], pooled, (((1,), (1,)), ((), ())),
        preferred_element_type=jnp.float32)
    o_ref[...] = logits_t + b_ref[...]


def _proj_body_aliased(prev_ref, c_ref, t_ref, w_ref, b_ref, o_ref):
    del prev_ref
    _proj_body(c_ref, t_ref, w_ref, b_ref, o_ref)


@functools.cache
def _make_proj(chunk):
    off = chunk * (_BC // _BLK)
    common = dict(
        grid=(_BC // _BLK,),
        out_specs=pl.BlockSpec((_VOCAB, _BLK), lambda i: (0, i + off)),
        out_shape=jax.ShapeDtypeStruct((_VOCAB, _B), jnp.float32),
        compiler_params=pltpu.CompilerParams(
            dimension_semantics=("arbitrary",)),
    )
    data_specs = [
        pl.BlockSpec((_BLK, _VPAD // 128, 128), lambda i: (i, 0, 0)),
        pl.BlockSpec((_VPAD, _DIM), lambda i: (0, 0)),
        pl.BlockSpec((_VOCAB, _DIM), lambda i: (0, 0)),
        pl.BlockSpec((_VOCAB, 1), lambda i: (0, 0)),
    ]
    if chunk == 0:
        return pl.pallas_call(_proj_body, in_specs=data_specs, **common)
    return pl.pallas_call(
        _proj_body_aliased,
        in_specs=[pl.BlockSpec(memory_space=pl.ANY)] + data_specs,
        input_output_aliases={0: 0},
        **common)


def kernel(x, table, W, b):
    x4 = (x.astype(jnp.int32).T
          .reshape(_L // 8, 8, _B // 128, 128).transpose(0, 2, 1, 3))
    table_p = jnp.pad(table, ((0, _VPAD - _VOCAB), (0, 0)))
    bcol = b.reshape(_VOCAB, 1)
    C = _make_count(0)(x4)
    logits_t = _make_proj(0)(C, table_p, W, bcol)
    for k in range(1, _NCHUNK):
        C = _make_count(k)(x4)
        logits_t = _make_proj(k)(logits_t, C, table_p, W, bcol)
    return logits_t.T
